# baseline (speedup 1.0000x reference)
import jax
import jax.numpy as jnp
from jax import lax
from jax.experimental import pallas as pl
from jax.experimental.pallas import tpu as pltpu
from jax.experimental.pallas import tpu_sc as plsc

NC = 2
NS = 16
NW = NC * NS
L = 16

BATCH_SIZE = 16384
N_ROWS = 1000000
N_FACT = 16
B_PER_W = BATCH_SIZE // NW
N_GROUPS = B_PER_W // L


def _body(uidx_hbm, iidx_hbm, ufacT_hbm, ifacT_hbm, out_hbm,
          uidx_v, iidx_v, blk_v, out_v, usem, isem):
    wid = lax.axis_index("s") * NC + lax.axis_index("c")
    base = wid * B_PER_W

    pltpu.sync_copy(uidx_hbm.at[pl.ds(base, B_PER_W)], uidx_v)
    pltpu.sync_copy(iidx_hbm.at[pl.ds(base, B_PER_W)], iidx_v)

    lane = lax.iota(jnp.int32, L)
    blk2d = blk_v.reshape(2 * L * N_FACT, 128)

    def fire_group(g, buf):
        ub = uidx_v[pl.ds(g * L, L)] >> 4
        ib = iidx_v[pl.ds(g * L, L)] >> 4
        for b in range(L):
            pltpu.async_copy(
                ufacT_hbm.at[:, :, pl.ds(ub[b] * 16, 16)],
                blk_v.at[buf, b, :, :, pl.ds(0, 16)], usem)
            pltpu.async_copy(
                ifacT_hbm.at[:, :, pl.ds(ib[b] * 16, 16)],
                blk_v.at[buf, b, :, :, pl.ds(16, 16)], isem)

    def drain_group(buf):
        for b in range(L):
            pltpu.make_async_copy(
                ufacT_hbm.at[:, :, pl.ds(0, 16)],
                blk_v.at[buf, b, :, :, pl.ds(0, 16)], usem).wait()
            pltpu.make_async_copy(
                ifacT_hbm.at[:, :, pl.ds(0, 16)],
                blk_v.at[buf, b, :, :, pl.ds(16, 16)], isem).wait()

    def compute_group(g, buf):
        uoff = uidx_v[pl.ds(g * L, L)] & 15
        ioff = (iidx_v[pl.ds(g * L, L)] & 15) + 16
        base_row = (buf * L + lane) * N_FACT
        acc = jnp.zeros((L,), jnp.float32)
        for k in range(N_FACT):
            rows = base_row + k
            uf = plsc.load_gather(blk2d, [rows, uoff])
            vf = plsc.load_gather(blk2d, [rows, ioff])
            acc = acc + uf * vf
        out_v[pl.ds(g * L, L)] = acc * 5.0

    fire_group(0, 0)

    def one_group(g, _):
        buf = lax.rem(g, 2)

        @pl.when(g < N_GROUPS - 1)
        def _():
            fire_group(g + 1, 1 - buf)

        drain_group(buf)
        compute_group(g, buf)
        return 0

    lax.fori_loop(0, N_GROUPS, one_group, 0)
    pltpu.sync_copy(out_v, out_hbm.at[pl.ds(base, B_PER_W)])


@jax.jit
def _mf_kernel(u_idx, i_idx, ufacT, ifacT):
    mesh = plsc.VectorSubcoreMesh(core_axis_name="c", subcore_axis_name="s")
    return pl.kernel(
        _body,
        out_type=jax.ShapeDtypeStruct((BATCH_SIZE,), jnp.float32),
        mesh=mesh,
        compiler_params=pltpu.CompilerParams(
            needs_layout_passes=False,
            disable_bounds_checks=True,
            disable_semaphore_checks=True,
            skip_device_barrier=True,
        ),
        scratch_types=[
            pltpu.VMEM((B_PER_W,), jnp.int32),
            pltpu.VMEM((B_PER_W,), jnp.int32),
            pltpu.VMEM((2, L, 2, 8, 128), jnp.float32),
            pltpu.VMEM((B_PER_W,), jnp.float32),
            pltpu.SemaphoreType.DMA,
            pltpu.SemaphoreType.DMA,
        ],
    )(u_idx, i_idx, ufacT, ifacT)


def kernel(user, item, user_factors, item_factors):
    ufT = user_factors.T.reshape(2, 8, N_ROWS)
    ifT = item_factors.T.reshape(2, 8, N_ROWS)
    return _mf_kernel(user - 1, item - 1, ufT, ifT)


# speedup vs baseline: 1.0285x; 1.0285x over previous
import jax
import jax.numpy as jnp
from jax import lax
from jax.experimental import pallas as pl
from jax.experimental.pallas import tpu as pltpu
from jax.experimental.pallas import tpu_sc as plsc

NC = 2
NS = 16
NW = NC * NS
L = 16

BATCH_SIZE = 16384
N_ROWS = 1000000
N_FACT = 16
B_PER_W = BATCH_SIZE // NW
N_GROUPS = B_PER_W // L


def _body(uidx_hbm, iidx_hbm, ufacT_hbm, ifacT_hbm, out_hbm,
          uidx_v, iidx_v, blk_v, out_v, dummy_hbm, usem, isem):
    wid = lax.axis_index("s") * NC + lax.axis_index("c")
    base = wid * B_PER_W

    pltpu.sync_copy(uidx_hbm.at[pl.ds(base, B_PER_W)], uidx_v)
    pltpu.sync_copy(iidx_hbm.at[pl.ds(base, B_PER_W)], iidx_v)

    lane = lax.iota(jnp.int32, L)
    blk2d = blk_v.reshape(2 * L * N_FACT, 128)

    def fire_group(g, buf):
        ub = uidx_v[pl.ds(g * L, L)] >> 4
        ib = iidx_v[pl.ds(g * L, L)] >> 4
        for b in range(L):
            pltpu.async_copy(
                ufacT_hbm.at[:, :, pl.ds(ub[b] * 16, 16)],
                blk_v.at[buf, b, :, :, pl.ds(0, 16)], usem)
            pltpu.async_copy(
                ifacT_hbm.at[:, :, pl.ds(ib[b] * 16, 16)],
                blk_v.at[buf, b, :, :, pl.ds(16, 16)], isem)

    def drain_group(buf):
        pltpu.make_async_copy(
            dummy_hbm.at[:, :, :, pl.ds(0, 16)],
            blk_v.at[buf, :, :, :, pl.ds(0, 16)], usem).wait()
        pltpu.make_async_copy(
            dummy_hbm.at[:, :, :, pl.ds(16, 16)],
            blk_v.at[buf, :, :, :, pl.ds(16, 16)], isem).wait()

    def compute_group(g, buf):
        uoff = uidx_v[pl.ds(g * L, L)] & 15
        ioff = (iidx_v[pl.ds(g * L, L)] & 15) + 16
        base_row = (buf * L + lane) * N_FACT
        acc = jnp.zeros((L,), jnp.float32)
        for k in range(N_FACT):
            rows = base_row + k
            uf = plsc.load_gather(blk2d, [rows, uoff])
            vf = plsc.load_gather(blk2d, [rows, ioff])
            acc = acc + uf * vf
        out_v[pl.ds(g * L, L)] = acc * 5.0

    fire_group(0, 0)

    def one_group(g, _):
        buf = lax.rem(g, 2)

        @pl.when(g < N_GROUPS - 1)
        def _():
            fire_group(g + 1, 1 - buf)

        drain_group(buf)
        compute_group(g, buf)
        return 0

    lax.fori_loop(0, N_GROUPS, one_group, 0)
    pltpu.sync_copy(out_v, out_hbm.at[pl.ds(base, B_PER_W)])


@jax.jit
def _mf_kernel(u_idx, i_idx, ufacT, ifacT):
    mesh = plsc.VectorSubcoreMesh(core_axis_name="c", subcore_axis_name="s")
    return pl.kernel(
        _body,
        out_type=jax.ShapeDtypeStruct((BATCH_SIZE,), jnp.float32),
        mesh=mesh,
        compiler_params=pltpu.CompilerParams(
            needs_layout_passes=False,
            disable_bounds_checks=True,
            disable_semaphore_checks=True,
            skip_device_barrier=True,
        ),
        scratch_types=[
            pltpu.VMEM((B_PER_W,), jnp.int32),
            pltpu.VMEM((B_PER_W,), jnp.int32),
            pltpu.VMEM((2, L, 2, 8, 128), jnp.float32),
            pltpu.VMEM((B_PER_W,), jnp.float32),
            pltpu.MemorySpace.HBM((L, 2, 8, 128), jnp.float32),
            pltpu.SemaphoreType.DMA,
            pltpu.SemaphoreType.DMA,
        ],
    )(u_idx, i_idx, ufacT, ifacT)


def kernel(user, item, user_factors, item_factors):
    ufT = user_factors.T.reshape(2, 8, N_ROWS)
    ifT = item_factors.T.reshape(2, 8, N_ROWS)
    return _mf_kernel(user - 1, item - 1, ufT, ifT)


# speedup vs baseline: 1.0342x; 1.0055x over previous
import jax
import jax.numpy as jnp
from jax import lax
from jax.experimental import pallas as pl
from jax.experimental.pallas import tpu as pltpu
from jax.experimental.pallas import tpu_sc as plsc

NC = 2
NS = 16
NW = NC * NS
L = 16

BATCH_SIZE = 16384
N_ROWS = 1000000
N_FACT = 16
B_PER_W = BATCH_SIZE // NW
N_GROUPS = B_PER_W // L


def _body(uidx_hbm, iidx_hbm, ufacT_hbm, ifacT_hbm, out_hbm,
          uidx_v, iidx_v, blk_v, out_v, dummy_hbm, usem, isem):
    wid = lax.axis_index("s") * NC + lax.axis_index("c")
    base = wid * B_PER_W

    pltpu.sync_copy(uidx_hbm.at[pl.ds(base, B_PER_W)], uidx_v)
    pltpu.sync_copy(iidx_hbm.at[pl.ds(base, B_PER_W)], iidx_v)

    lane = lax.iota(jnp.int32, L)
    blk2d = blk_v.reshape(2 * L * N_FACT, 128)

    def fire_group(g, buf):
        ub = (uidx_v[pl.ds(g * L, L)] - 1) >> 4
        ib = (iidx_v[pl.ds(g * L, L)] - 1) >> 4
        for b in range(L):
            pltpu.async_copy(
                ufacT_hbm.at[:, :, pl.ds(ub[b] * 16, 16)],
                blk_v.at[buf, b, :, :, pl.ds(0, 16)], usem)
            pltpu.async_copy(
                ifacT_hbm.at[:, :, pl.ds(ib[b] * 16, 16)],
                blk_v.at[buf, b, :, :, pl.ds(16, 16)], isem)

    def drain_group(buf):
        pltpu.make_async_copy(
            dummy_hbm.at[:, :, :, pl.ds(0, 16)],
            blk_v.at[buf, :, :, :, pl.ds(0, 16)], usem).wait()
        pltpu.make_async_copy(
            dummy_hbm.at[:, :, :, pl.ds(16, 16)],
            blk_v.at[buf, :, :, :, pl.ds(16, 16)], isem).wait()

    def compute_group(g, buf):
        uoff = (uidx_v[pl.ds(g * L, L)] - 1) & 15
        ioff = ((iidx_v[pl.ds(g * L, L)] - 1) & 15) + 16
        base_row = (buf * L + lane) * N_FACT
        acc = jnp.zeros((L,), jnp.float32)
        for k in range(N_FACT):
            rows = base_row + k
            uf = plsc.load_gather(blk2d, [rows, uoff])
            vf = plsc.load_gather(blk2d, [rows, ioff])
            acc = acc + uf * vf
        out_v[pl.ds(g * L, L)] = acc * 5.0

    fire_group(0, 0)

    def one_group(g, _):
        buf = lax.rem(g, 2)

        @pl.when(g < N_GROUPS - 1)
        def _():
            fire_group(g + 1, 1 - buf)

        drain_group(buf)
        compute_group(g, buf)
        return 0

    lax.fori_loop(0, N_GROUPS, one_group, 0)
    pltpu.sync_copy(out_v, out_hbm.at[pl.ds(base, B_PER_W)])


@jax.jit
def _mf_kernel(u_idx, i_idx, ufacT, ifacT):
    mesh = plsc.VectorSubcoreMesh(core_axis_name="c", subcore_axis_name="s")
    return pl.kernel(
        _body,
        out_type=jax.ShapeDtypeStruct((BATCH_SIZE,), jnp.float32),
        mesh=mesh,
        compiler_params=pltpu.CompilerParams(
            needs_layout_passes=False,
            disable_bounds_checks=True,
            disable_semaphore_checks=True,
            skip_device_barrier=True,
        ),
        scratch_types=[
            pltpu.VMEM((B_PER_W,), jnp.int32),
            pltpu.VMEM((B_PER_W,), jnp.int32),
            pltpu.VMEM((2, L, 2, 8, 128), jnp.float32),
            pltpu.VMEM((B_PER_W,), jnp.float32),
            pltpu.MemorySpace.HBM((L, 2, 8, 128), jnp.float32),
            pltpu.SemaphoreType.DMA,
            pltpu.SemaphoreType.DMA,
        ],
    )(u_idx, i_idx, ufacT, ifacT)


def kernel(user, item, user_factors, item_factors):
    ufT = user_factors.T.reshape(2, 8, N_ROWS)
    ifT = item_factors.T.reshape(2, 8, N_ROWS)
    return _mf_kernel(user, item, ufT, ifT)


# speedup vs baseline: 1.0368x; 1.0026x over previous
import jax
import jax.numpy as jnp
from jax import lax
from jax.experimental import pallas as pl
from jax.experimental.pallas import tpu as pltpu
from jax.experimental.pallas import tpu_sc as plsc

NC = 2
NS = 16
NW = NC * NS
L = 16

BATCH_SIZE = 16384
N_ROWS = 1000000
N_FACT = 16
B_PER_W = BATCH_SIZE // NW
N_GROUPS = B_PER_W // L


def _body(uidx_hbm, iidx_hbm, ufacT_hbm, ifacT_hbm, out_hbm,
          uidx_v, iidx_v, blk_v, out_v, dummy_hbm, usem, isem):
    wid = lax.axis_index("s") * NC + lax.axis_index("c")
    base = wid * B_PER_W

    pltpu.sync_copy(uidx_hbm.at[pl.ds(base, B_PER_W)], uidx_v)
    pltpu.sync_copy(iidx_hbm.at[pl.ds(base, B_PER_W)], iidx_v)

    lane = lax.iota(jnp.int32, L)
    blk2d = blk_v.reshape(2 * L * N_FACT, 128)

    def fire_group(g, buf):
        ub = (uidx_v[pl.ds(g * L, L)] - 1) >> 4
        ib = (iidx_v[pl.ds(g * L, L)] - 1) >> 4
        for b in range(L):
            pltpu.async_copy(
                ufacT_hbm.at[:, :, pl.ds(ub[b] * 16, 16)],
                blk_v.at[buf, b, :, :, pl.ds(0, 16)], usem)
            pltpu.async_copy(
                ifacT_hbm.at[:, :, pl.ds(ib[b] * 16, 16)],
                blk_v.at[buf, b, :, :, pl.ds(16, 16)], isem)

    def drain_group(buf):
        pltpu.make_async_copy(
            dummy_hbm.at[:, :, :, pl.ds(0, 16)],
            blk_v.at[buf, :, :, :, pl.ds(0, 16)], usem).wait()
        pltpu.make_async_copy(
            dummy_hbm.at[:, :, :, pl.ds(16, 16)],
            blk_v.at[buf, :, :, :, pl.ds(16, 16)], isem).wait()

    def compute_group(g, buf):
        uoff = (uidx_v[pl.ds(g * L, L)] - 1) & 15
        ioff = ((iidx_v[pl.ds(g * L, L)] - 1) & 15) + 16
        base_row = (buf * L + lane) * N_FACT
        acc = jnp.zeros((L,), jnp.float32)
        for k in range(N_FACT):
            rows = base_row + k
            uf = plsc.load_gather(blk2d, [rows, uoff])
            vf = plsc.load_gather(blk2d, [rows, ioff])
            acc = acc + uf * vf
        out_v[pl.ds(g * L, L)] = acc * 5.0

    fire_group(0, 0)

    def one_group(g, _):
        buf = lax.rem(g, 2)

        @pl.when(g < N_GROUPS - 1)
        def _():
            fire_group(g + 1, 1 - buf)

        drain_group(buf)
        compute_group(g, buf)
        return 0

    lax.fori_loop(0, N_GROUPS, one_group, 0)
    pltpu.sync_copy(out_v, out_hbm.at[pl.ds(base, B_PER_W)])


@jax.jit
def _mf_kernel(u_idx, i_idx, ufacT, ifacT):
    mesh = plsc.VectorSubcoreMesh(core_axis_name="c", subcore_axis_name="s")
    return pl.kernel(
        _body,
        out_type=jax.ShapeDtypeStruct((BATCH_SIZE,), jnp.float32),
        mesh=mesh,
        compiler_params=pltpu.CompilerParams(
            needs_layout_passes=False,
            disable_bounds_checks=True,
            disable_semaphore_checks=True,
        ),
        scratch_types=[
            pltpu.VMEM((B_PER_W,), jnp.int32),
            pltpu.VMEM((B_PER_W,), jnp.int32),
            pltpu.VMEM((2, L, 2, 8, 128), jnp.float32),
            pltpu.VMEM((B_PER_W,), jnp.float32),
            pltpu.MemorySpace.HBM((L, 2, 8, 128), jnp.float32),
            pltpu.SemaphoreType.DMA,
            pltpu.SemaphoreType.DMA,
        ],
    )(u_idx, i_idx, ufacT, ifacT)


def kernel(user, item, user_factors, item_factors):
    ufT = user_factors.T.reshape(2, 8, N_ROWS)
    ifT = item_factors.T.reshape(2, 8, N_ROWS)
    return _mf_kernel(user, item, ufT, ifT)


# speedup vs baseline: 1.0407x; 1.0037x over previous
import jax
import jax.numpy as jnp
from jax import lax
from jax.experimental import pallas as pl
from jax.experimental.pallas import tpu as pltpu
from jax.experimental.pallas import tpu_sc as plsc

NC = 2
NS = 16
NW = NC * NS
L = 16

BATCH_SIZE = 16384
N_ROWS = 1000000
N_FACT = 16
B_PER_W = BATCH_SIZE // NW
N_GROUPS = B_PER_W // L


def _body(uidx_hbm, iidx_hbm, ufacT_hbm, ifacT_hbm, out_hbm,
          uidx_v, iidx_v, blk_v, out_v, dummy_hbm, usem, isem, usem2, isem2):
    wid = lax.axis_index("s") * NC + lax.axis_index("c")
    base = wid * B_PER_W

    pltpu.sync_copy(uidx_hbm.at[pl.ds(base, B_PER_W)], uidx_v)
    pltpu.sync_copy(iidx_hbm.at[pl.ds(base, B_PER_W)], iidx_v)

    lane = lax.iota(jnp.int32, L)
    blk2d = blk_v.reshape(2 * L * N_FACT, 128)

    def fire_group(g, buf):
        ub = (uidx_v[pl.ds(g * L, L)] - 1) >> 4
        ib = (iidx_v[pl.ds(g * L, L)] - 1) >> 4
        for b in range(L):
            pltpu.async_copy(
                ufacT_hbm.at[:, :, pl.ds(ub[b] * 16, 16)],
                blk_v.at[buf, b, :, :, pl.ds(0, 16)],
                usem if b % 2 == 0 else usem2)
            pltpu.async_copy(
                ifacT_hbm.at[:, :, pl.ds(ib[b] * 16, 16)],
                blk_v.at[buf, b, :, :, pl.ds(16, 16)],
                isem if b % 2 == 0 else isem2)

    def drain_group(buf):
        pltpu.make_async_copy(
            dummy_hbm.at[pl.ds(0, 8), :, :, pl.ds(0, 16)],
            blk_v.at[buf, pl.ds(0, 8), :, :, pl.ds(0, 16)], usem).wait()
        pltpu.make_async_copy(
            dummy_hbm.at[pl.ds(0, 8), :, :, pl.ds(0, 16)],
            blk_v.at[buf, pl.ds(0, 8), :, :, pl.ds(0, 16)], usem2).wait()
        pltpu.make_async_copy(
            dummy_hbm.at[pl.ds(0, 8), :, :, pl.ds(16, 16)],
            blk_v.at[buf, pl.ds(0, 8), :, :, pl.ds(16, 16)], isem).wait()
        pltpu.make_async_copy(
            dummy_hbm.at[pl.ds(0, 8), :, :, pl.ds(16, 16)],
            blk_v.at[buf, pl.ds(0, 8), :, :, pl.ds(16, 16)], isem2).wait()

    def compute_group(g, buf):
        uoff = (uidx_v[pl.ds(g * L, L)] - 1) & 15
        ioff = ((iidx_v[pl.ds(g * L, L)] - 1) & 15) + 16
        base_row = (buf * L + lane) * N_FACT
        acc = jnp.zeros((L,), jnp.float32)
        for k in range(N_FACT):
            rows = base_row + k
            uf = plsc.load_gather(blk2d, [rows, uoff])
            vf = plsc.load_gather(blk2d, [rows, ioff])
            acc = acc + uf * vf
        out_v[pl.ds(g * L, L)] = acc * 5.0

    fire_group(0, 0)

    def one_group(g, _):
        buf = lax.rem(g, 2)

        @pl.when(g < N_GROUPS - 1)
        def _():
            fire_group(g + 1, 1 - buf)

        drain_group(buf)
        compute_group(g, buf)
        return 0

    lax.fori_loop(0, N_GROUPS, one_group, 0)
    pltpu.sync_copy(out_v, out_hbm.at[pl.ds(base, B_PER_W)])


@jax.jit
def _mf_kernel(u_idx, i_idx, ufacT, ifacT):
    mesh = plsc.VectorSubcoreMesh(core_axis_name="c", subcore_axis_name="s")
    return pl.kernel(
        _body,
        out_type=jax.ShapeDtypeStruct((BATCH_SIZE,), jnp.float32),
        mesh=mesh,
        compiler_params=pltpu.CompilerParams(
            needs_layout_passes=False,
            disable_bounds_checks=True,
            disable_semaphore_checks=True,
        ),
        scratch_types=[
            pltpu.VMEM((B_PER_W,), jnp.int32),
            pltpu.VMEM((B_PER_W,), jnp.int32),
            pltpu.VMEM((2, L, 2, 8, 128), jnp.float32),
            pltpu.VMEM((B_PER_W,), jnp.float32),
            pltpu.MemorySpace.HBM((L, 2, 8, 128), jnp.float32),
            pltpu.SemaphoreType.DMA,
            pltpu.SemaphoreType.DMA,
            pltpu.SemaphoreType.DMA,
            pltpu.SemaphoreType.DMA,
        ],
    )(u_idx, i_idx, ufacT, ifacT)


def kernel(user, item, user_factors, item_factors):
    ufT = user_factors.T.reshape(2, 8, N_ROWS)
    ifT = item_factors.T.reshape(2, 8, N_ROWS)
    return _mf_kernel(user, item, ufT, ifT)
